# baseline (speedup 1.0000x reference)
import functools

import jax
import jax.numpy as jnp
from jax import lax
from jax.experimental import pallas as pl
from jax.experimental.pallas import tpu as pltpu
from jax.experimental.pallas import tpu_sc as plsc

VOCAB = 1000000
DIM = 64
BATCH = 16384
NNEG = 20
NPAIR = NNEG + 1

_INFO = plsc.get_sparse_core_info()
NC = _INFO.num_cores
NS = _INFO.num_subcores
NW = NC * NS
B_PER_W = BATCH // NW
C = 32
NCHUNK = B_PER_W // C
NEG_PER_CHUNK = C * NNEG
NNEG_W = B_PER_W * NNEG
SPLIT = 1 << 19

_mesh = plsc.VectorSubcoreMesh(core_axis_name="c", subcore_axis_name="s")


@functools.partial(
    pl.kernel,
    out_type=jax.ShapeDtypeStruct((BATCH * NPAIR,), jnp.float32),
    mesh=_mesh,
    compiler_params=pltpu.CompilerParams(needs_layout_passes=False,
                                         use_tc_tiling_on_sc=True),
    scratch_types=[
        pltpu.VMEM((B_PER_W,), jnp.int32),
        pltpu.VMEM((B_PER_W,), jnp.int32),
        pltpu.VMEM((NNEG_W,), jnp.int32),
        pltpu.VMEM((B_PER_W,), jnp.int32),
        pltpu.VMEM((B_PER_W,), jnp.int32),
        pltpu.VMEM((NNEG_W,), jnp.int32),
        pltpu.VMEM((C, 128), jnp.float32),
        pltpu.VMEM((C, 128), jnp.float32),
        pltpu.VMEM((NEG_PER_CHUNK, 128), jnp.float32),
        pltpu.VMEM((C * NPAIR,), jnp.float32),
        pltpu.SemaphoreType.DMA,
    ],
)
def _sc_dots(cen_hbm, ctx_hbm, neg_hbm, win_hbm, wout_hbm, out_hbm,
             cen_h, ctx_h, neg_h, cen_o, ctx_o, neg_o,
             cen_v, ctx_v, neg_v, out_v, sem):
    wid = lax.axis_index("s") * NC + lax.axis_index("c")

    pltpu.sync_copy(cen_hbm.at[pl.ds(wid * B_PER_W, B_PER_W)], cen_h)
    pltpu.sync_copy(ctx_hbm.at[pl.ds(wid * B_PER_W, B_PER_W)], ctx_h)
    pltpu.sync_copy(neg_hbm.at[pl.ds(wid * NNEG_W, NNEG_W)], neg_h)

    def split_ids(n, buf, offs):
        def body(j, _):
            v = buf[pl.ds(j * 16, 16)]
            offs[pl.ds(j * 16, 16)] = lax.shift_right_logical(v, 19) * 64
            buf[pl.ds(j * 16, 16)] = v & (SPLIT - 1)
            return 0
        lax.fori_loop(0, n // 16, body, 0)

    split_ids(B_PER_W, cen_h, cen_o)
    split_ids(B_PER_W, ctx_h, ctx_o)
    split_ids(NNEG_W, neg_h, neg_o)

    lane = lax.broadcasted_iota(jnp.int32, (16,), 0)
    last_lane = lane == 15

    def splat_elem(offs, idx):
        vec = offs[pl.ds((idx // 16) * 16, 16)]
        return lax.gather(
            vec, jnp.broadcast_to(idx % 16, (16,))[:, None],
            lax.GatherDimensionNumbers(offset_dims=(),
                                       collapsed_slice_dims=(0,),
                                       start_index_map=(0,)),
            slice_sizes=(1,),
            mode=lax.GatherScatterMode.PROMISE_IN_BOUNDS)

    def chunk_body(t, _):
        base = wid * B_PER_W + t * C

        cps = [
            pltpu.async_copy(win_hbm.at[cen_h.at[pl.ds(t * C, C)]],
                             cen_v, sem),
            pltpu.async_copy(wout_hbm.at[ctx_h.at[pl.ds(t * C, C)]],
                             ctx_v, sem),
        ]
        for q in range(NEG_PER_CHUNK // 128):
            cps.append(pltpu.async_copy(
                wout_hbm.at[neg_h.at[pl.ds(t * NEG_PER_CHUNK + q * 128, 128)]],
                neg_v.at[pl.ds(q * 128, 128)], sem))
        for cp in cps:
            cp.wait()

        def elem_body(i, _):
            coff = splat_elem(cen_o, t * C + i) + lane
            c = [plsc.load_gather(cen_v, [jnp.broadcast_to(i, (16,)),
                                          coff + k * 16])
                 for k in range(DIM // 16)]

            def emit_dot(buf, row, off_splat, slot):
                col = off_splat + lane
                y = [plsc.load_gather(buf, [jnp.broadcast_to(row, (16,)),
                                            col + k * 16])
                     for k in range(DIM // 16)]
                p = (c[0] * y[0] + c[1] * y[1]) + (c[2] * y[2] + c[3] * y[3])
                s = plsc.cumsum(p)
                plsc.store_scatter(out_v, [jnp.full((16,), slot, jnp.int32)],
                                   s, mask=last_lane)

            emit_dot(ctx_v, i, splat_elem(ctx_o, t * C + i), i * NPAIR)
            for n in range(NNEG):
                r = i * NNEG + n
                emit_dot(neg_v, r, splat_elem(neg_o, t * NEG_PER_CHUNK + r),
                         i * NPAIR + (n + 1))
            return 0

        lax.fori_loop(0, C, elem_body, 0)
        pltpu.sync_copy(out_v, out_hbm.at[pl.ds(base * NPAIR, C * NPAIR)])
        return 0

    lax.fori_loop(0, NCHUNK, chunk_body, 0)


_TBLK = 256


def _tc_transpose_body(lo_ref, hi_ref, out_ref):
    lo = jnp.transpose(lo_ref[...], (1, 0))
    hi = jnp.transpose(hi_ref[...], (1, 0))
    out_ref[...] = jnp.concatenate([lo, hi], axis=1)


def _tc_relayout(wt):
    nblk = SPLIT // _TBLK
    jmax = (VOCAB + _TBLK - 1) // _TBLK - 1
    return pl.pallas_call(
        _tc_transpose_body,
        grid=(nblk,),
        in_specs=[pl.BlockSpec((DIM, _TBLK), lambda j: (0, j)),
                  pl.BlockSpec((DIM, _TBLK),
                               lambda j: (0, jnp.minimum(j + nblk, jmax)))],
        out_specs=pl.BlockSpec((_TBLK, 128), lambda j: (j, 0)),
        out_shape=jax.ShapeDtypeStruct((SPLIT, 128), jnp.float32),
    )(wt, wt)


def _tc_loss_body(dots_ref, out_ref):
    x = dots_ref[...]
    rows, cols = x.shape
    flat = (lax.broadcasted_iota(jnp.int32, (rows, cols), 0) * cols
            + lax.broadcasted_iota(jnp.int32, (rows, cols), 1))
    v = jnp.where(flat % NPAIR == 0, x, -x)
    ls = -(jnp.maximum(-v, 0.0) + jnp.log1p(jnp.exp(-jnp.abs(v))))
    out_ref[...] = jnp.reshape(-jnp.sum(ls) / BATCH, (1, 1))


def kernel(center_ids, context_ids, negative_ids, W_in, W_out):
    neg_flat = negative_ids.reshape(BATCH * NNEG)
    win2 = _tc_relayout(jnp.transpose(W_in))
    wout2 = _tc_relayout(jnp.transpose(W_out))
    dots = _sc_dots(center_ids, context_ids, neg_flat, win2, wout2)
    dots2d = dots.reshape(BATCH * NPAIR // 128, 128)
    loss = pl.pallas_call(
        _tc_loss_body,
        out_shape=jax.ShapeDtypeStruct((1, 1), jnp.float32),
    )(dots2d)
    return loss[0, 0]


# speedup vs baseline: 3.4547x; 3.4547x over previous
import functools

import jax
import jax.numpy as jnp
from jax import lax
from jax.experimental import pallas as pl
from jax.experimental.pallas import tpu as pltpu
from jax.experimental.pallas import tpu_sc as plsc

VOCAB = 1000000
DIM = 64
BATCH = 16384
NNEG = 20
NPAIR = NNEG + 1

_INFO = plsc.get_sparse_core_info()
NC = _INFO.num_cores
NS = _INFO.num_subcores
NW = NC * NS
B_PER_W = BATCH // NW
C = 32
NCHUNK = B_PER_W // C
NEG_PER_CHUNK = C * NNEG
NNEG_W = B_PER_W * NNEG
SPLIT = 1 << 19

_mesh = plsc.VectorSubcoreMesh(core_axis_name="c", subcore_axis_name="s")


@functools.partial(
    pl.kernel,
    out_type=jax.ShapeDtypeStruct((BATCH * NPAIR,), jnp.float32),
    mesh=_mesh,
    compiler_params=pltpu.CompilerParams(needs_layout_passes=False,
                                         use_tc_tiling_on_sc=True),
    scratch_types=[
        pltpu.VMEM((B_PER_W,), jnp.int32),
        pltpu.VMEM((B_PER_W,), jnp.int32),
        pltpu.VMEM((NNEG_W,), jnp.int32),
        pltpu.VMEM((B_PER_W,), jnp.int32),
        pltpu.VMEM((B_PER_W,), jnp.int32),
        pltpu.VMEM((NNEG_W,), jnp.int32),
        pltpu.VMEM((C, 128), jnp.float32),
        pltpu.VMEM((C, 128), jnp.float32),
        pltpu.VMEM((NEG_PER_CHUNK, 128), jnp.float32),
        pltpu.VMEM((C * NPAIR,), jnp.float32),
        pltpu.SemaphoreType.DMA,
    ],
)
def _sc_dots(cen_hbm, ctx_hbm, neg_hbm, win_hbm, wout_hbm, out_hbm,
             cen_h, ctx_h, neg_h, cen_o, ctx_o, neg_o,
             cen_v, ctx_v, neg_v, out_v, sem):
    wid = lax.axis_index("s") * NC + lax.axis_index("c")

    pltpu.sync_copy(cen_hbm.at[pl.ds(wid * B_PER_W, B_PER_W)], cen_h)
    pltpu.sync_copy(ctx_hbm.at[pl.ds(wid * B_PER_W, B_PER_W)], ctx_h)
    pltpu.sync_copy(neg_hbm.at[pl.ds(wid * NNEG_W, NNEG_W)], neg_h)

    def split_ids(n, buf, offs):
        def body(j, _):
            v = buf[pl.ds(j * 16, 16)]
            offs[pl.ds(j * 16, 16)] = lax.shift_right_logical(v, 19) * 64
            buf[pl.ds(j * 16, 16)] = v & (SPLIT - 1)
            return 0
        lax.fori_loop(0, n // 16, body, 0)

    split_ids(B_PER_W, cen_h, cen_o)
    split_ids(B_PER_W, ctx_h, ctx_o)
    split_ids(NNEG_W, neg_h, neg_o)

    lane = lax.broadcasted_iota(jnp.int32, (16,), 0)
    last_lane = lane == 15

    def splat_elem(offs, idx):
        vec = offs[pl.ds((idx // 16) * 16, 16)]
        return lax.gather(
            vec, jnp.broadcast_to(idx % 16, (16,))[:, None],
            lax.GatherDimensionNumbers(offset_dims=(),
                                       collapsed_slice_dims=(0,),
                                       start_index_map=(0,)),
            slice_sizes=(1,),
            mode=lax.GatherScatterMode.PROMISE_IN_BOUNDS)

    def chunk_body(t, _):
        base = wid * B_PER_W + t * C

        cps = [
            pltpu.async_copy(win_hbm.at[cen_h.at[pl.ds(t * C, C)]],
                             cen_v, sem),
            pltpu.async_copy(wout_hbm.at[ctx_h.at[pl.ds(t * C, C)]],
                             ctx_v, sem),
        ]
        for q in range(NEG_PER_CHUNK // 128):
            cps.append(pltpu.async_copy(
                wout_hbm.at[neg_h.at[pl.ds(t * NEG_PER_CHUNK + q * 128, 128)]],
                neg_v.at[pl.ds(q * 128, 128)], sem))
        for cp in cps:
            cp.wait()

        def elem_body(i, _):
            coff = splat_elem(cen_o, t * C + i) + lane
            c = [plsc.load_gather(cen_v, [jnp.broadcast_to(i, (16,)),
                                          coff + k * 16])
                 for k in range(DIM // 16)]

            def emit_dot(buf, row, off_splat, slot):
                col = off_splat + lane
                y = [plsc.load_gather(buf, [jnp.broadcast_to(row, (16,)),
                                            col + k * 16])
                     for k in range(DIM // 16)]
                p = (c[0] * y[0] + c[1] * y[1]) + (c[2] * y[2] + c[3] * y[3])
                s = plsc.cumsum(p)
                plsc.store_scatter(out_v, [jnp.full((16,), slot, jnp.int32)],
                                   s, mask=last_lane)

            emit_dot(ctx_v, i, splat_elem(ctx_o, t * C + i), i * NPAIR)
            for n in range(NNEG):
                r = i * NNEG + n
                emit_dot(neg_v, r, splat_elem(neg_o, t * NEG_PER_CHUNK + r),
                         i * NPAIR + (n + 1))
            return 0

        lax.fori_loop(0, C, elem_body, 0)
        pltpu.sync_copy(out_v, out_hbm.at[pl.ds(base * NPAIR, C * NPAIR)])
        return 0

    lax.fori_loop(0, NCHUNK, chunk_body, 0)


_TBLK = 8192


def _tc_transpose_body(lo_ref, hi_ref, out_ref):
    lo = jnp.transpose(lo_ref[...], (1, 0))
    hi = jnp.transpose(hi_ref[...], (1, 0))
    out_ref[...] = jnp.concatenate([lo, hi], axis=1)


def _tc_relayout(wt):
    nblk = SPLIT // _TBLK
    jmax = (VOCAB + _TBLK - 1) // _TBLK - 1
    return pl.pallas_call(
        _tc_transpose_body,
        grid=(nblk,),
        in_specs=[pl.BlockSpec((DIM, _TBLK), lambda j: (0, j)),
                  pl.BlockSpec((DIM, _TBLK),
                               lambda j: (0, jnp.minimum(j + nblk, jmax)))],
        out_specs=pl.BlockSpec((_TBLK, 128), lambda j: (j, 0)),
        out_shape=jax.ShapeDtypeStruct((SPLIT, 128), jnp.float32),
    )(wt, wt)


def _tc_loss_body(dots_ref, out_ref):
    x = dots_ref[...]
    rows, cols = x.shape
    flat = (lax.broadcasted_iota(jnp.int32, (rows, cols), 0) * cols
            + lax.broadcasted_iota(jnp.int32, (rows, cols), 1))
    v = jnp.where(flat % NPAIR == 0, x, -x)
    ls = -(jnp.maximum(-v, 0.0) + jnp.log1p(jnp.exp(-jnp.abs(v))))
    out_ref[...] = jnp.reshape(-jnp.sum(ls) / BATCH, (1, 1))


def kernel(center_ids, context_ids, negative_ids, W_in, W_out):
    neg_flat = negative_ids.reshape(BATCH * NNEG)
    win2 = _tc_relayout(jnp.transpose(W_in))
    wout2 = _tc_relayout(jnp.transpose(W_out))
    dots = _sc_dots(center_ids, context_ids, neg_flat, win2, wout2)
    dots2d = dots.reshape(BATCH * NPAIR // 128, 128)
    loss = pl.pallas_call(
        _tc_loss_body,
        out_shape=jax.ShapeDtypeStruct((1, 1), jnp.float32),
    )(dots2d)
    return loss[0, 0]


# speedup vs baseline: 3.7665x; 1.0903x over previous
import functools

import jax
import jax.numpy as jnp
from jax import lax
from jax.experimental import pallas as pl
from jax.experimental.pallas import tpu as pltpu
from jax.experimental.pallas import tpu_sc as plsc

VOCAB = 1000000
DIM = 64
BATCH = 16384
NNEG = 20
NPAIR = NNEG + 1

_INFO = plsc.get_sparse_core_info()
NC = _INFO.num_cores
NS = _INFO.num_subcores
NW = NC * NS
B_PER_W = BATCH // NW
C = 16
NCHUNK = B_PER_W // C
NEG_PER_CHUNK = C * NNEG
NNEG_W = B_PER_W * NNEG
SPLIT = 1 << 19

_mesh = plsc.VectorSubcoreMesh(core_axis_name="c", subcore_axis_name="s")


@functools.partial(
    pl.kernel,
    out_type=jax.ShapeDtypeStruct((BATCH * NPAIR,), jnp.float32),
    mesh=_mesh,
    compiler_params=pltpu.CompilerParams(needs_layout_passes=False,
                                         use_tc_tiling_on_sc=True),
    scratch_types=[
        pltpu.VMEM((B_PER_W,), jnp.int32),
        pltpu.VMEM((B_PER_W,), jnp.int32),
        pltpu.VMEM((NNEG_W,), jnp.int32),
        pltpu.VMEM((B_PER_W,), jnp.int32),
        pltpu.VMEM((B_PER_W,), jnp.int32),
        pltpu.VMEM((NNEG_W,), jnp.int32),
        pltpu.VMEM((C, 128), jnp.float32),
        pltpu.VMEM((C, 128), jnp.float32),
        pltpu.VMEM((NEG_PER_CHUNK, 128), jnp.float32),
        pltpu.VMEM((C, 128), jnp.float32),
        pltpu.VMEM((C, 128), jnp.float32),
        pltpu.VMEM((NEG_PER_CHUNK, 128), jnp.float32),
        pltpu.VMEM((C * NPAIR,), jnp.float32),
        pltpu.SemaphoreType.DMA,
        pltpu.SemaphoreType.DMA,
    ],
)
def _sc_dots(cen_hbm, ctx_hbm, neg_hbm, win_hbm, wout_hbm, out_hbm,
             cen_h, ctx_h, neg_h, cen_o, ctx_o, neg_o,
             cen_v0, ctx_v0, neg_v0, cen_v1, ctx_v1, neg_v1,
             out_v, sem0, sem1):
    wid = lax.axis_index("s") * NC + lax.axis_index("c")

    pltpu.sync_copy(cen_hbm.at[pl.ds(wid * B_PER_W, B_PER_W)], cen_h)
    pltpu.sync_copy(ctx_hbm.at[pl.ds(wid * B_PER_W, B_PER_W)], ctx_h)
    pltpu.sync_copy(neg_hbm.at[pl.ds(wid * NNEG_W, NNEG_W)], neg_h)

    def split_ids(n, buf, offs):
        def body(j, _):
            v = buf[pl.ds(j * 16, 16)]
            offs[pl.ds(j * 16, 16)] = lax.shift_right_logical(v, 19) * 64
            buf[pl.ds(j * 16, 16)] = v & (SPLIT - 1)
            return 0
        lax.fori_loop(0, n // 16, body, 0)

    split_ids(B_PER_W, cen_h, cen_o)
    split_ids(B_PER_W, ctx_h, ctx_o)
    split_ids(NNEG_W, neg_h, neg_o)

    lane = lax.broadcasted_iota(jnp.int32, (16,), 0)
    last_lane = lane == 15

    def splat_elem(offs, idx):
        vec = offs[pl.ds((idx // 16) * 16, 16)]
        return lax.gather(
            vec, jnp.broadcast_to(idx % 16, (16,))[:, None],
            lax.GatherDimensionNumbers(offset_dims=(),
                                       collapsed_slice_dims=(0,),
                                       start_index_map=(0,)),
            slice_sizes=(1,),
            mode=lax.GatherScatterMode.PROMISE_IN_BOUNDS)

    neg_slices = [(0, 128), (128, 128), (256, 64)][:(NEG_PER_CHUNK + 127)
                                                   // 128]

    def issue(t, cen_b, ctx_b, neg_b, sm):
        pltpu.async_copy(win_hbm.at[cen_h.at[pl.ds(t * C, C)]], cen_b, sm)
        pltpu.async_copy(wout_hbm.at[ctx_h.at[pl.ds(t * C, C)]], ctx_b, sm)
        for o, l in neg_slices:
            pltpu.async_copy(
                wout_hbm.at[neg_h.at[pl.ds(t * NEG_PER_CHUNK + o, l)]],
                neg_b.at[pl.ds(o, l)], sm)

    def drain(cen_b, ctx_b, neg_b, sm):
        pltpu.make_async_copy(win_hbm.at[pl.ds(0, C)], cen_b, sm).wait()
        pltpu.make_async_copy(wout_hbm.at[pl.ds(0, C)], ctx_b, sm).wait()
        for o, l in neg_slices:
            pltpu.make_async_copy(wout_hbm.at[pl.ds(0, l)],
                                  neg_b.at[pl.ds(o, l)], sm).wait()

    def compute(t, cen_b, ctx_b, neg_b):
        base = wid * B_PER_W + t * C

        def elem_body(i, _):
            coff = splat_elem(cen_o, t * C + i) + lane
            c = [plsc.load_gather(cen_b, [jnp.broadcast_to(i, (16,)),
                                          coff + k * 16])
                 for k in range(DIM // 16)]

            def emit_dot(buf, row, off_splat, slot):
                col = off_splat + lane
                y = [plsc.load_gather(buf, [jnp.broadcast_to(row, (16,)),
                                            col + k * 16])
                     for k in range(DIM // 16)]
                p = (c[0] * y[0] + c[1] * y[1]) + (c[2] * y[2] + c[3] * y[3])
                s = plsc.cumsum(p)
                plsc.store_scatter(out_v, [jnp.full((16,), slot, jnp.int32)],
                                   s, mask=last_lane)

            emit_dot(ctx_b, i, splat_elem(ctx_o, t * C + i), i * NPAIR)
            for n in range(NNEG):
                r = i * NNEG + n
                emit_dot(neg_b, r, splat_elem(neg_o, t * NEG_PER_CHUNK + r),
                         i * NPAIR + (n + 1))
            return 0

        lax.fori_loop(0, C, elem_body, 0)
        pltpu.sync_copy(out_v, out_hbm.at[pl.ds(base * NPAIR, C * NPAIR)])

    bufs = ((cen_v0, ctx_v0, neg_v0, sem0), (cen_v1, ctx_v1, neg_v1, sem1))
    issue(0, *bufs[0])

    def pair_body(tt, _):
        for p in range(2):
            t = tt * 2 + p
            cb, xb, nb, sm = bufs[p]
            drain(cb, xb, nb, sm)
            issue(jnp.minimum(t + 1, NCHUNK - 1), *bufs[1 - p])
            compute(t, cb, xb, nb)
        return 0

    lax.fori_loop(0, NCHUNK // 2, pair_body, 0)
    drain(*bufs[0])


_TBLK = 8192


def _tc_transpose_body(lo_ref, hi_ref, out_ref):
    lo = jnp.transpose(lo_ref[...], (1, 0))
    hi = jnp.transpose(hi_ref[...], (1, 0))
    out_ref[...] = jnp.concatenate([lo, hi], axis=1)


def _tc_relayout(wt):
    nblk = SPLIT // _TBLK
    jmax = (VOCAB + _TBLK - 1) // _TBLK - 1
    return pl.pallas_call(
        _tc_transpose_body,
        grid=(nblk,),
        in_specs=[pl.BlockSpec((DIM, _TBLK), lambda j: (0, j)),
                  pl.BlockSpec((DIM, _TBLK),
                               lambda j: (0, jnp.minimum(j + nblk, jmax)))],
        out_specs=pl.BlockSpec((_TBLK, 128), lambda j: (j, 0)),
        out_shape=jax.ShapeDtypeStruct((SPLIT, 128), jnp.float32),
    )(wt, wt)


def _tc_loss_body(dots_ref, out_ref):
    x = dots_ref[...]
    rows, cols = x.shape
    flat = (lax.broadcasted_iota(jnp.int32, (rows, cols), 0) * cols
            + lax.broadcasted_iota(jnp.int32, (rows, cols), 1))
    v = jnp.where(flat % NPAIR == 0, x, -x)
    ls = -(jnp.maximum(-v, 0.0) + jnp.log1p(jnp.exp(-jnp.abs(v))))
    out_ref[...] = jnp.reshape(-jnp.sum(ls) / BATCH, (1, 1))


def kernel(center_ids, context_ids, negative_ids, W_in, W_out):
    neg_flat = negative_ids.reshape(BATCH * NNEG)
    win2 = _tc_relayout(jnp.transpose(W_in))
    wout2 = _tc_relayout(jnp.transpose(W_out))
    dots = _sc_dots(center_ids, context_ids, neg_flat, win2, wout2)
    dots2d = dots.reshape(BATCH * NPAIR // 128, 128)
    loss = pl.pallas_call(
        _tc_loss_body,
        out_shape=jax.ShapeDtypeStruct((1, 1), jnp.float32),
    )(dots2d)
    return loss[0, 0]
